# baseline (speedup 1.0000x reference)
import functools

import jax
import jax.numpy as jnp
from jax import lax
from jax.experimental import pallas as pl
from jax.experimental.pallas import tpu as pltpu
from jax.experimental.pallas import tpu_sc as plsc

N = 10000
E = 320000
D = 128

NC = 2
NS = 16
NW = NC * NS
EPW = E // NW
CH = 80
NCHUNK = EPW // CH
N_PAD = 10240
RPT = N_PAD // NS
WB = 128

@functools.lru_cache(maxsize=None)
def _mesh():
    return plsc.VectorSubcoreMesh(
        core_axis_name="c", subcore_axis_name="s", num_cores=NC,
        num_subcores=NS,
    )


def _zero_vmem(ref, rows, cols):
    zer = jnp.zeros((16,), jnp.float32)

    def body(i, _):
        r = i // (cols // 16)
        cidx = i % (cols // 16)
        ref[r, pl.ds(cidx * 16, 16)] = zer
        return 0

    lax.fori_loop(0, rows * (cols // 16), body, 0)


def _sc_gather_scatter(table, src_idx, dst_idx):
    def body(table_h, src_h, dst_h, out_h, src_v, dst_v, rows_v,
             acc_sh, sem):
        c = lax.axis_index("c")
        s = lax.axis_index("s")
        wid = c * NS + s

        _zero_vmem(rows_v, CH, D)

        def zb2(m, _):
            pltpu.sync_copy(rows_v, acc_sh.at[pl.ds(s * RPT + m * CH, CH)])
            return 0

        lax.fori_loop(0, RPT // CH, zb2, 0)
        plsc.subcore_barrier()

        pltpu.sync_copy(src_h.at[wid], src_v)
        pltpu.sync_copy(dst_h.at[wid], dst_v)

        def chunk(g, _):
            pltpu.async_copy(table_h.at[src_v.at[g]], rows_v, sem).wait()
            pltpu.sync_copy(rows_v, acc_sh.at[dst_v.at[g]], add=True)
            return 0

        lax.fori_loop(0, NCHUNK, chunk, 0)
        plsc.subcore_barrier()

        def wb(m, _):
            base = s * RPT + m * CH
            pltpu.sync_copy(acc_sh.at[pl.ds(base, CH)], rows_v)
            pltpu.sync_copy(rows_v, out_h.at[c, pl.ds(base, CH)])
            return 0

        lax.fori_loop(0, RPT // CH, wb, 0)

    f = pl.kernel(
        body,
        out_type=jax.ShapeDtypeStruct((NC, N_PAD, D), jnp.float32),
        mesh=_mesh(),
        scratch_types=[
            pltpu.VMEM((NCHUNK, CH), jnp.int32),
            pltpu.VMEM((NCHUNK, CH), jnp.int32),
            pltpu.VMEM((CH, D), jnp.float32),
            pltpu.VMEM_SHARED((N_PAD, D), jnp.float32),
            pltpu.SemaphoreType.DMA,
        ],
    )
    return f(table, src_idx, dst_idx)


BN = 1000


def _tc0(degp, x):
    def body(degp_ref, x_ref, dis_ref, u_ref):
        deg = degp_ref[0, :, 0:1] + degp_ref[1, :, 0:1]
        good = deg > 0.0
        dis = jnp.where(good, lax.rsqrt(jnp.where(good, deg, 1.0)), 0.0)
        dis_ref[...] = dis
        u_ref[...] = x_ref[...] * dis

    return pl.pallas_call(
        body,
        grid=(N // BN,),
        in_specs=[
            pl.BlockSpec((NC, BN, D), lambda i: (0, i, 0)),
            pl.BlockSpec((BN, D), lambda i: (i, 0)),
        ],
        out_specs=[
            pl.BlockSpec((BN, 1), lambda i: (i, 0)),
            pl.BlockSpec((BN, D), lambda i: (i, 0)),
        ],
        out_shape=[
            jax.ShapeDtypeStruct((N, 1), jnp.float32),
            jax.ShapeDtypeStruct((N, D), jnp.float32),
        ],
    )(degp, x)


def _tc_mid(sp, z, dis, W):
    dout = W.shape[2]

    def body(sp_ref, z_ref, dis_ref, w_ref, outa_ref, u_ref):
        dis = dis_ref[...]
        tx1 = -(sp_ref[0] + sp_ref[1]) * dis
        z = z_ref[...]
        outa_ref[...] = (
            jnp.dot(z, w_ref[0], preferred_element_type=jnp.float32)
            + jnp.dot(tx1, w_ref[1], preferred_element_type=jnp.float32)
        )
        u_ref[...] = tx1 * dis

    return pl.pallas_call(
        body,
        grid=(N // BN,),
        in_specs=[
            pl.BlockSpec((NC, BN, D), lambda i: (0, i, 0)),
            pl.BlockSpec((BN, D), lambda i: (i, 0)),
            pl.BlockSpec((BN, 1), lambda i: (i, 0)),
            pl.BlockSpec((3, D, dout), lambda i: (0, 0, 0)),
        ],
        out_specs=[
            pl.BlockSpec((BN, dout), lambda i: (i, 0)),
            pl.BlockSpec((BN, D), lambda i: (i, 0)),
        ],
        out_shape=[
            jax.ShapeDtypeStruct((N, dout), jnp.float32),
            jax.ShapeDtypeStruct((N, D), jnp.float32),
        ],
    )(sp, z, dis, W)


def _tc_post1(sp, z, outa, dis, W, b):
    def body(sp_ref, z_ref, outa_ref, dis_ref, w_ref, b_ref, h_ref, u_ref):
        dis = dis_ref[...]
        tx2 = -2.0 * (sp_ref[0] + sp_ref[1]) * dis - z_ref[...]
        h = outa_ref[...] + jnp.dot(
            tx2, w_ref[2], preferred_element_type=jnp.float32) + b_ref[...]
        h = jnp.maximum(h, 0.0)
        h_ref[...] = h
        u_ref[...] = h * dis

    return pl.pallas_call(
        body,
        grid=(N // BN,),
        in_specs=[
            pl.BlockSpec((NC, BN, D), lambda i: (0, i, 0)),
            pl.BlockSpec((BN, D), lambda i: (i, 0)),
            pl.BlockSpec((BN, D), lambda i: (i, 0)),
            pl.BlockSpec((BN, 1), lambda i: (i, 0)),
            pl.BlockSpec((3, D, D), lambda i: (0, 0, 0)),
            pl.BlockSpec((1, D), lambda i: (0, 0)),
        ],
        out_specs=[
            pl.BlockSpec((BN, D), lambda i: (i, 0)),
            pl.BlockSpec((BN, D), lambda i: (i, 0)),
        ],
        out_shape=[
            jax.ShapeDtypeStruct((N, D), jnp.float32),
            jax.ShapeDtypeStruct((N, D), jnp.float32),
        ],
    )(sp, z, outa, dis, W, b)


def _tc_post2(sp, z, outb, dis, W, b):
    dout = W.shape[2]

    def body(sp_ref, z_ref, outb_ref, dis_ref, w_ref, b_ref, o_ref):
        dis = dis_ref[...]
        tx2 = -2.0 * (sp_ref[0] + sp_ref[1]) * dis - z_ref[...]
        logits = outb_ref[...] + jnp.dot(
            tx2, w_ref[2], preferred_element_type=jnp.float32) + b_ref[...]
        m = jnp.max(logits, axis=1, keepdims=True)
        sh = logits - m
        lse = jnp.log(jnp.sum(jnp.exp(sh), axis=1, keepdims=True))
        o_ref[...] = sh - lse

    return pl.pallas_call(
        body,
        grid=(N // BN,),
        in_specs=[
            pl.BlockSpec((NC, BN, D), lambda i: (0, i, 0)),
            pl.BlockSpec((BN, D), lambda i: (i, 0)),
            pl.BlockSpec((BN, dout), lambda i: (i, 0)),
            pl.BlockSpec((BN, 1), lambda i: (i, 0)),
            pl.BlockSpec((3, D, dout), lambda i: (0, 0, 0)),
            pl.BlockSpec((1, dout), lambda i: (0, 0)),
        ],
        out_specs=pl.BlockSpec((BN, dout), lambda i: (i, 0)),
        out_shape=jax.ShapeDtypeStruct((N, dout), jnp.float32),
    )(sp, z, outb, dis, W, b)


def kernel(x, edge_index, W1, b1, W2, b2):
    row = edge_index[0].reshape(NW, NCHUNK, CH)
    col = edge_index[1].reshape(NW, NCHUNK, CH)

    ones_t = jnp.ones((8, D), jnp.float32)
    zeros_i = jnp.zeros((NW, NCHUNK, CH), jnp.int32)
    degp = _sc_gather_scatter(ones_t, zeros_i, row)
    dis, u0 = _tc0(degp, x)

    s1 = _sc_gather_scatter(u0, row, col)
    outa, u1 = _tc_mid(s1, x, dis, W1)
    s2 = _sc_gather_scatter(u1, row, col)
    h, u2 = _tc_post1(s2, x, outa, dis, W1, b1.reshape(1, -1))

    s3 = _sc_gather_scatter(u2, row, col)
    outb, u3 = _tc_mid(s3, h, dis, W2)
    s4 = _sc_gather_scatter(u3, row, col)
    return _tc_post2(s4, h, outb, dis, W2, b2.reshape(1, -1))
